# Initial kernel scaffold; baseline (speedup 1.0000x reference)
import functools
import math

import jax
import jax.numpy as jnp
from jax.experimental import pallas as pl
from jax.experimental.pallas import tpu as pltpu

BT = 256
HC = 1024


def _gating_body(x_ref, wg_ref, trust_ref, w_ref):
    T = x_ref.shape[0]
    E = wg_ref.shape[0]
    gates = jax.lax.dot_general(
        x_ref[...], wg_ref[...], (((1,), (1,)), ((), ())),
        preferred_element_type=jnp.float32)
    trust = trust_ref[0, :]
    sig = 1.0 / (1.0 + jnp.exp(-trust))
    cur = gates * sig
    lane = jax.lax.broadcasted_iota(jnp.int32, (T, E), 1)
    vals = []
    idxs = []
    for _ in range(4):
        m = jnp.max(cur, axis=1, keepdims=True)
        sel = cur == m
        idx = jnp.min(jnp.where(sel, lane, E), axis=1, keepdims=True)
        vals.append(m)
        idxs.append(idx)
        cur = jnp.where(lane == idx, -1e30, cur)
    v0 = vals[0]
    exps = [jnp.exp(v - v0) for v in vals]
    denom = exps[0] + exps[1] + exps[2] + exps[3]
    w = jnp.zeros((T, E), jnp.float32)
    for e_k, i_k in zip(exps, idxs):
        w = w + jnp.where(lane == i_k, e_k / denom, 0.0)
    w_ref[...] = w


def _gelu(h):
    return 0.5 * h * (1.0 + jax.lax.erf(h * (1.0 / math.sqrt(2.0))))


def _ffn_body(x_ref, w1_ref, b1_ref, w2_ref, b2_ref, w_ref, out_ref):
    e = pl.program_id(0)
    hc = pl.program_id(1)
    tb = pl.program_id(2)
    row0 = tb * BT
    xb = x_ref[pl.ds(row0, BT), :].astype(jnp.bfloat16)
    w1 = w1_ref[0].astype(jnp.bfloat16)
    h = jax.lax.dot_general(
        xb, w1, (((1,), (1,)), ((), ())),
        preferred_element_type=jnp.float32)
    h = h + b1_ref[0, 0]
    h = _gelu(h).astype(jnp.bfloat16)
    w2 = w2_ref[0].astype(jnp.bfloat16)
    part = jax.lax.dot_general(
        h, w2, (((1,), (1,)), ((), ())),
        preferred_element_type=jnp.float32)
    part = part + jnp.where(hc == 0, 1.0, 0.0) * b2_ref[0]
    wfull = w_ref[pl.ds(row0, BT), :]
    esel = jax.lax.broadcasted_iota(jnp.int32, wfull.shape, 1) == e
    wsel = jnp.sum(jnp.where(esel, wfull, 0.0), axis=1, keepdims=True)
    contrib = wsel * part
    first = jnp.logical_and(e == 0, hc == 0)

    @pl.when(first)
    def _():
        out_ref[pl.ds(row0, BT), :] = contrib

    @pl.when(jnp.logical_not(first))
    def _():
        out_ref[pl.ds(row0, BT), :] += contrib


def kernel(x, Wg, W1, b1, W2, b2, trust_scores):
    Bq, Sq, Dq = x.shape
    x_flat = x.reshape(-1, Dq)
    T = x_flat.shape[0]
    E, H, D = W1.shape

    weights = pl.pallas_call(
        _gating_body,
        out_shape=jax.ShapeDtypeStruct((T, E), jnp.float32),
        in_specs=[
            pl.BlockSpec((T, D), lambda: (0, 0)),
            pl.BlockSpec((E, D), lambda: (0, 0)),
            pl.BlockSpec((1, E), lambda: (0, 0)),
        ],
        out_specs=pl.BlockSpec((T, E), lambda: (0, 0)),
    )(x_flat, Wg, trust_scores.reshape(1, E))

    n_hc = H // HC
    n_tb = T // BT
    b1r = b1.reshape(E, n_hc, 1, HC)
    b2r = b2.reshape(E, 1, D)

    out = pl.pallas_call(
        _ffn_body,
        grid=(E, n_hc, n_tb),
        out_shape=jax.ShapeDtypeStruct((T, D), jnp.float32),
        in_specs=[
            pl.BlockSpec((T, D), lambda e, hc, tb: (0, 0)),
            pl.BlockSpec((1, HC, D), lambda e, hc, tb: (e, hc, 0)),
            pl.BlockSpec((1, 1, 1, HC), lambda e, hc, tb: (e, hc, 0, 0)),
            pl.BlockSpec((1, D, HC), lambda e, hc, tb: (e, 0, hc)),
            pl.BlockSpec((1, 1, D), lambda e, hc, tb: (e, 0, 0)),
            pl.BlockSpec((T, E), lambda e, hc, tb: (0, 0)),
        ],
        out_specs=pl.BlockSpec((T, D), lambda e, hc, tb: (0, 0)),
        compiler_params=pltpu.CompilerParams(
            dimension_semantics=("arbitrary", "arbitrary", "arbitrary"),
        ),
    )(x_flat, W1, b1r, W2, b2r, weights)

    return out.reshape(Bq, Sq, Dq)


# speedup vs baseline: 2.1643x; 2.1643x over previous
import functools
import math

import jax
import jax.numpy as jnp
from jax import lax
from jax.experimental import pallas as pl
from jax.experimental.pallas import tpu as pltpu
from jax.experimental.pallas import tpu_sc as plsc

K = 4
BT = 256
HC = 1024
EPI = 8


def _gating_body(x_ref, wg_ref, trust_ref, idx_ref, prob_ref):
    T = x_ref.shape[0]
    E = wg_ref.shape[0]
    gates = lax.dot_general(
        x_ref[...], wg_ref[...], (((1,), (1,)), ((), ())),
        preferred_element_type=jnp.float32)
    trust = trust_ref[0, :]
    sig = 1.0 / (1.0 + jnp.exp(-trust))
    cur = gates * sig
    lane = lax.broadcasted_iota(jnp.int32, (T, E), 1)
    vals, idxs = [], []
    for _ in range(K):
        m = jnp.max(cur, axis=1, keepdims=True)
        idx = jnp.min(jnp.where(cur == m, lane, E), axis=1, keepdims=True)
        vals.append(m)
        idxs.append(idx)
        cur = jnp.where(lane == idx, -1e30, cur)
    v0 = vals[0]
    exps = [jnp.exp(v - v0) for v in vals]
    denom = exps[0] + exps[1] + exps[2] + exps[3]
    kl = lax.broadcasted_iota(jnp.int32, (T, K), 1)
    iout = jnp.zeros((T, K), jnp.int32)
    pout = jnp.zeros((T, K), jnp.float32)
    for kk in range(K):
        iout = jnp.where(kl == kk, idxs[kk], iout)
        pout = jnp.where(kl == kk, exps[kk] / denom, pout)
    idx_ref[...] = iout
    prob_ref[...] = pout


def _gating(x_flat, Wg, trust):
    T, D = x_flat.shape
    E = Wg.shape[0]
    return pl.pallas_call(
        _gating_body,
        out_shape=(jax.ShapeDtypeStruct((T, K), jnp.int32),
                   jax.ShapeDtypeStruct((T, K), jnp.float32)),
        in_specs=[
            pl.BlockSpec((T, D), lambda: (0, 0)),
            pl.BlockSpec((E, D), lambda: (0, 0)),
            pl.BlockSpec((1, E), lambda: (0, 0)),
        ],
        out_specs=(pl.BlockSpec((T, K), lambda: (0, 0)),
                   pl.BlockSpec((T, K), lambda: (0, 0))),
    )(x_flat, Wg, trust.reshape(1, E))


def _dispatch(topk_idx, topk_probs, E, nblk):
    T = topk_idx.shape[0]
    TK = T * K
    flat_e = topk_idx.reshape(-1)
    order = jnp.argsort(flat_e, stable=True).astype(jnp.int32)
    sorted_tok = (order // K).astype(jnp.int32)
    sorted_prob = topk_probs.reshape(-1)[order]
    counts = jnp.bincount(flat_e, length=E).astype(jnp.int32)
    offs = jnp.cumsum(counts) - counts
    nb = (counts + BT - 1) // BT
    cnb = jnp.cumsum(nb)
    bloff = cnb - nb
    bi = jnp.arange(nblk, dtype=jnp.int32)
    eid = jnp.searchsorted(cnb, bi, side='right').astype(jnp.int32)
    eid = jnp.minimum(eid, E - 1)
    j = bi - bloff[eid]
    start = offs[eid] + j * BT
    blen = jnp.clip(counts[eid] - j * BT, 0, BT)
    s = jnp.arange(nblk * BT, dtype=jnp.int32)
    sb, r = s // BT, s % BT
    pos = jnp.minimum(start[sb] + r, TK - 1)
    valid = r < blen[sb]
    slot_tok = jnp.where(valid, sorted_tok[pos], 0).astype(jnp.int32)
    slot_prob = jnp.where(valid, sorted_prob[pos], 0.0)
    return eid, slot_tok, slot_prob


def _sc_gather(x_flat, slot_tok, ns):
    T, D = x_flat.shape
    info = plsc.get_sparse_core_info()
    nw = info.num_cores * info.num_subcores
    b_per_w = ns // nw
    chunk = 64
    n_chunks = b_per_w // chunk
    mesh = plsc.VectorSubcoreMesh(core_axis_name="c", subcore_axis_name="s")

    @functools.partial(
        pl.kernel, mesh=mesh,
        out_type=jax.ShapeDtypeStruct((ns, D), jnp.float32),
        scratch_types=[
            pltpu.VMEM((b_per_w,), jnp.int32),
            pltpu.VMEM((chunk, D), jnp.float32),
            pltpu.SemaphoreType.DMA,
        ],
    )
    def gather(x_hbm, tok_hbm, out_hbm, idx_v, rows_v, sem):
        wid = lax.axis_index("s") * info.num_cores + lax.axis_index("c")
        base = wid * b_per_w
        pltpu.sync_copy(tok_hbm.at[pl.ds(base, b_per_w)], idx_v)
        for c in range(n_chunks):
            pltpu.async_copy(
                x_hbm.at[idx_v.at[pl.ds(c * chunk, chunk)]], rows_v, sem
            ).wait()
            pltpu.sync_copy(rows_v, out_hbm.at[pl.ds(base + c * chunk, chunk)])

    return gather(x_flat, slot_tok)


def _gelu(h):
    return 0.5 * h * (1.0 + lax.erf(h * (1.0 / math.sqrt(2.0))))


def _ffn_body(eid_ref, xs_ref, w1_ref, b1_ref, w2_ref, b2_ref,
              tok_ref, prob_ref, out_ref, acc_ref):
    b = pl.program_id(0)
    nsteps = pl.num_programs(0)
    nblk = nsteps - EPI
    T = acc_ref.shape[0]
    H = w1_ref.shape[1]

    @pl.when(b == 0)
    def _():
        acc_ref[...] = jnp.zeros_like(acc_ref)

    @pl.when(b < nblk)
    def _():
        xb = xs_ref[...].astype(jnp.bfloat16)
        eo = None
        for hc in range(H // HC):
            w1c = w1_ref[0, hc * HC:(hc + 1) * HC, :]
            h = lax.dot_general(xb, w1c, (((1,), (1,)), ((), ())),
                                preferred_element_type=jnp.float32)
            h = h + b1_ref[0, 0, hc * HC:(hc + 1) * HC]
            h = _gelu(h).astype(jnp.bfloat16)
            w2c = w2_ref[0, :, hc * HC:(hc + 1) * HC]
            p = lax.dot_general(h, w2c, (((1,), (1,)), ((), ())),
                                preferred_element_type=jnp.float32)
            eo = p if eo is None else eo + p
        eo = eo + b2_ref[0, 0]
        prob_col = prob_ref[0]
        contrib = (prob_col * eo).astype(jnp.bfloat16)
        tok_row = tok_ref[0]
        tok_b = tok_row + jnp.zeros((T, 1), jnp.int32)
        diff = lax.broadcasted_iota(jnp.int32, tok_b.shape, 0) - tok_b
        onehot_t = (1.0 - jnp.minimum(
            jnp.abs(diff).astype(jnp.float32), 1.0)).astype(jnp.bfloat16)
        acc_ref[...] += lax.dot_general(
            onehot_t, contrib, (((1,), (0,)), ((), ())),
            preferred_element_type=jnp.float32)

    @pl.when(b >= nblk)
    def _():
        out_ref[...] = acc_ref[pl.ds((b - nblk) * (T // EPI), T // EPI), :]


def _ffn(xs, W1bf, b1, W2bf, b2, eid, slot_tok, slot_prob, T, nblk):
    E, H, D = W1bf.shape
    nsteps = nblk + EPI
    grid_spec = pltpu.PrefetchScalarGridSpec(
        num_scalar_prefetch=1,
        grid=(nsteps,),
        in_specs=[
            pl.BlockSpec((BT, D), lambda b, er: (jnp.minimum(b, er.shape[0] - EPI - 1), 0)),
            pl.BlockSpec((1, H, D), lambda b, er: (er[b], 0, 0)),
            pl.BlockSpec((1, 1, H), lambda b, er: (er[b], 0, 0)),
            pl.BlockSpec((1, D, H), lambda b, er: (er[b], 0, 0)),
            pl.BlockSpec((1, 1, D), lambda b, er: (er[b], 0, 0)),
            pl.BlockSpec((1, 1, BT), lambda b, er: (jnp.minimum(b, er.shape[0] - EPI - 1), 0, 0)),
            pl.BlockSpec((1, BT, 1), lambda b, er: (jnp.minimum(b, er.shape[0] - EPI - 1), 0, 0)),
        ],
        out_specs=pl.BlockSpec(
            (T // EPI, D),
            lambda b, er: (jnp.clip(b - (er.shape[0] - EPI), 0, EPI - 1), 0)),
        scratch_shapes=[pltpu.VMEM((T, D), jnp.float32)],
    )
    eid_pad = jnp.concatenate(
        [eid, jnp.full((EPI,), E - 1, jnp.int32)])
    return pl.pallas_call(
        _ffn_body,
        grid_spec=grid_spec,
        out_shape=jax.ShapeDtypeStruct((T, D), jnp.float32),
        compiler_params=pltpu.CompilerParams(
            dimension_semantics=("arbitrary",),
        ),
    )(eid_pad, xs, W1bf, b1.reshape(E, 1, H), W2bf, b2.reshape(E, 1, D),
      slot_tok.reshape(nblk, 1, BT), slot_prob.reshape(nblk, BT, 1))


def kernel(x, Wg, W1, b1, W2, b2, trust_scores):
    Bq, Sq, Dq = x.shape
    x_flat = x.reshape(-1, Dq)
    T = x_flat.shape[0]
    E, H, D = W1.shape
    nblk = (T * K) // BT + E
    ns = nblk * BT

    topk_idx, topk_probs = _gating(x_flat, Wg, trust_scores)
    eid, slot_tok, slot_prob = _dispatch(topk_idx, topk_probs, E, nblk)
    xs = _sc_gather(x_flat, slot_tok, ns)
    out = _ffn(xs, W1.astype(jnp.bfloat16), b1, W2.astype(jnp.bfloat16),
               b2, eid, slot_tok, slot_prob, T, nblk)
    return out.reshape(Bq, Sq, Dq)


# speedup vs baseline: 2.7966x; 1.2922x over previous
import math

import jax
import jax.numpy as jnp
from jax import lax
from jax.experimental import pallas as pl
from jax.experimental.pallas import tpu as pltpu

K = 4
BT = 256
HC = 1024
EPI = 8


def _gating_body(x_ref, wg_ref, trust_ref, idx_ref, prob_ref):
    T = x_ref.shape[0]
    E = wg_ref.shape[0]
    gates = lax.dot_general(
        x_ref[...], wg_ref[...], (((1,), (1,)), ((), ())),
        preferred_element_type=jnp.float32)
    trust = trust_ref[0, :]
    sig = 1.0 / (1.0 + jnp.exp(-trust))
    cur = gates * sig
    lane = lax.broadcasted_iota(jnp.int32, (T, E), 1)
    vals, idxs = [], []
    for _ in range(K):
        m = jnp.max(cur, axis=1, keepdims=True)
        idx = jnp.min(jnp.where(cur == m, lane, E), axis=1, keepdims=True)
        vals.append(m)
        idxs.append(idx)
        cur = jnp.where(lane == idx, -1e30, cur)
    v0 = vals[0]
    exps = [jnp.exp(v - v0) for v in vals]
    denom = exps[0] + exps[1] + exps[2] + exps[3]
    kl = lax.broadcasted_iota(jnp.int32, (T, K), 1)
    iout = jnp.zeros((T, K), jnp.int32)
    pout = jnp.zeros((T, K), jnp.float32)
    for kk in range(K):
        iout = jnp.where(kl == kk, idxs[kk], iout)
        pout = jnp.where(kl == kk, exps[kk] / denom, pout)
    idx_ref[...] = iout
    prob_ref[...] = pout


def _gating(x_flat, Wg, trust):
    T, D = x_flat.shape
    E = Wg.shape[0]
    return pl.pallas_call(
        _gating_body,
        out_shape=(jax.ShapeDtypeStruct((T, K), jnp.int32),
                   jax.ShapeDtypeStruct((T, K), jnp.float32)),
        in_specs=[
            pl.BlockSpec((T, D), lambda: (0, 0)),
            pl.BlockSpec((E, D), lambda: (0, 0)),
            pl.BlockSpec((1, E), lambda: (0, 0)),
        ],
        out_specs=(pl.BlockSpec((T, K), lambda: (0, 0)),
                   pl.BlockSpec((T, K), lambda: (0, 0))),
    )(x_flat, Wg, trust.reshape(1, E))


def _dispatch(topk_idx, topk_probs, E, nblk):
    T = topk_idx.shape[0]
    TK = T * K
    flat_e = topk_idx.reshape(-1)
    order = jnp.argsort(flat_e, stable=True).astype(jnp.int32)
    sorted_tok = (order // K).astype(jnp.int32)
    sorted_prob = topk_probs.reshape(-1)[order]
    counts = jnp.bincount(flat_e, length=E).astype(jnp.int32)
    offs = jnp.cumsum(counts) - counts
    nb = (counts + BT - 1) // BT
    cnb = jnp.cumsum(nb)
    bloff = cnb - nb
    bi = jnp.arange(nblk, dtype=jnp.int32)
    eid = jnp.searchsorted(cnb, bi, side='right').astype(jnp.int32)
    eid = jnp.minimum(eid, E - 1)
    j = bi - bloff[eid]
    start = offs[eid] + j * BT
    blen = jnp.clip(counts[eid] - j * BT, 0, BT)
    s = jnp.arange(nblk * BT, dtype=jnp.int32)
    sb, r = s // BT, s % BT
    pos = jnp.minimum(start[sb] + r, TK - 1)
    valid = r < blen[sb]
    slot_tok = jnp.where(valid, sorted_tok[pos], 0).astype(jnp.int32)
    slot_prob = jnp.where(valid, sorted_prob[pos], 0.0)
    return eid, blen.astype(jnp.int32), slot_tok, slot_prob


def _gelu(h):
    return 0.5 * h * (1.0 + lax.erf(h * (1.0 / math.sqrt(2.0))))


def _ffn_body(eid_ref, blen_ref, xbf_ref, w1_ref, b1_ref, w2_ref, b2_ref,
              tokr_ref, tokc_ref, prob_ref, out_ref, acc_ref):
    b = pl.program_id(0)
    nsteps = pl.num_programs(0)
    nblk = nsteps - EPI
    T = acc_ref.shape[0]
    H = w1_ref.shape[1]

    @pl.when(b == 0)
    def _():
        acc_ref[...] = jnp.zeros_like(acc_ref)

    active = jnp.logical_and(b < nblk, blen_ref[jnp.minimum(b, nblk - 1)] > 0)

    @pl.when(active)
    def _():
        tok_col = tokc_ref[0]
        diff_g = (lax.broadcasted_iota(jnp.int32, (BT, T), 1) - tok_col)
        onehot_g = (1.0 - jnp.minimum(
            jnp.abs(diff_g).astype(jnp.float32), 1.0)).astype(jnp.bfloat16)
        xb = lax.dot_general(
            onehot_g, xbf_ref[...], (((1,), (0,)), ((), ())),
            preferred_element_type=jnp.float32).astype(jnp.bfloat16)
        eo = None
        for hc in range(H // HC):
            w1c = w1_ref[0, hc * HC:(hc + 1) * HC, :]
            h = lax.dot_general(xb, w1c, (((1,), (1,)), ((), ())),
                                preferred_element_type=jnp.float32)
            h = h + b1_ref[0, 0, hc * HC:(hc + 1) * HC]
            h = _gelu(h).astype(jnp.bfloat16)
            w2c = w2_ref[0, :, hc * HC:(hc + 1) * HC]
            p = lax.dot_general(h, w2c, (((1,), (1,)), ((), ())),
                                preferred_element_type=jnp.float32)
            eo = p if eo is None else eo + p
        eo = eo + b2_ref[0, 0]
        prob_col = prob_ref[0]
        contrib = (prob_col * eo).astype(jnp.bfloat16)
        tok_row = tokr_ref[0]
        tok_b = tok_row + jnp.zeros((T, 1), jnp.int32)
        diff_s = lax.broadcasted_iota(jnp.int32, tok_b.shape, 0) - tok_b
        onehot_t = (1.0 - jnp.minimum(
            jnp.abs(diff_s).astype(jnp.float32), 1.0)).astype(jnp.bfloat16)
        acc_ref[...] += lax.dot_general(
            onehot_t, contrib, (((1,), (0,)), ((), ())),
            preferred_element_type=jnp.float32)

    @pl.when(b >= nblk)
    def _():
        out_ref[...] = acc_ref[pl.ds((b - nblk) * (T // EPI), T // EPI), :]


def _ffn(xbf, W1bf, b1, W2bf, b2, eid, blen, slot_tok, slot_prob, T, nblk):
    E, H, D = W1bf.shape
    nsteps = nblk + EPI
    grid_spec = pltpu.PrefetchScalarGridSpec(
        num_scalar_prefetch=2,
        grid=(nsteps,),
        in_specs=[
            pl.BlockSpec((T, D), lambda b, er, lr: (0, 0)),
            pl.BlockSpec((1, H, D), lambda b, er, lr: (er[b], 0, 0)),
            pl.BlockSpec((1, 1, H), lambda b, er, lr: (er[b], 0, 0)),
            pl.BlockSpec((1, D, H), lambda b, er, lr: (er[b], 0, 0)),
            pl.BlockSpec((1, 1, D), lambda b, er, lr: (er[b], 0, 0)),
            pl.BlockSpec((1, 1, BT), lambda b, er, lr: (jnp.minimum(b, er.shape[0] - EPI - 1), 0, 0)),
            pl.BlockSpec((1, BT, 1), lambda b, er, lr: (jnp.minimum(b, er.shape[0] - EPI - 1), 0, 0)),
            pl.BlockSpec((1, BT, 1), lambda b, er, lr: (jnp.minimum(b, er.shape[0] - EPI - 1), 0, 0)),
        ],
        out_specs=pl.BlockSpec(
            (T // EPI, D),
            lambda b, er, lr: (jnp.clip(b - (er.shape[0] - EPI), 0, EPI - 1), 0)),
        scratch_shapes=[pltpu.VMEM((T, D), jnp.float32)],
    )
    eid_pad = jnp.concatenate(
        [eid, jnp.full((EPI,), E - 1, jnp.int32)])
    return pl.pallas_call(
        _ffn_body,
        grid_spec=grid_spec,
        out_shape=jax.ShapeDtypeStruct((T, D), jnp.float32),
        compiler_params=pltpu.CompilerParams(
            dimension_semantics=("arbitrary",),
        ),
    )(eid_pad, blen, xbf, W1bf, b1.reshape(E, 1, H), W2bf,
      b2.reshape(E, 1, D), slot_tok.reshape(nblk, 1, BT),
      slot_tok.reshape(nblk, BT, 1), slot_prob.reshape(nblk, BT, 1))


def kernel(x, Wg, W1, b1, W2, b2, trust_scores):
    Bq, Sq, Dq = x.shape
    x_flat = x.reshape(-1, Dq)
    T = x_flat.shape[0]
    E, H, D = W1.shape
    nblk = (T * K) // BT + E

    topk_idx, topk_probs = _gating(x_flat, Wg, trust_scores)
    eid, blen, slot_tok, slot_prob = _dispatch(topk_idx, topk_probs, E, nblk)
    out = _ffn(x_flat.astype(jnp.bfloat16), W1.astype(jnp.bfloat16), b1,
               W2.astype(jnp.bfloat16), b2, eid, blen, slot_tok, slot_prob,
               T, nblk)
    return out.reshape(Bq, Sq, Dq)


# speedup vs baseline: 3.1703x; 1.1336x over previous
import math

import jax
import jax.numpy as jnp
from jax import lax
from jax.experimental import pallas as pl
from jax.experimental.pallas import tpu as pltpu

K = 4
BT = 256
HC = 1024
NHC = 4
EPI = 8


def _gating_body(x_ref, wg_ref, trust_ref, idx_ref, prob_ref):
    T = x_ref.shape[0]
    E = wg_ref.shape[0]
    gates = lax.dot_general(
        x_ref[...], wg_ref[...], (((1,), (1,)), ((), ())),
        preferred_element_type=jnp.float32)
    trust = trust_ref[0, :]
    sig = 1.0 / (1.0 + jnp.exp(-trust))
    cur = gates * sig
    lane = lax.broadcasted_iota(jnp.int32, (T, E), 1)
    vals, idxs = [], []
    for _ in range(K):
        m = jnp.max(cur, axis=1, keepdims=True)
        idx = jnp.min(jnp.where(cur == m, lane, E), axis=1, keepdims=True)
        vals.append(m)
        idxs.append(idx)
        cur = jnp.where(lane == idx, -1e30, cur)
    v0 = vals[0]
    exps = [jnp.exp(v - v0) for v in vals]
    denom = exps[0] + exps[1] + exps[2] + exps[3]
    kl = lax.broadcasted_iota(jnp.int32, (T, K), 1)
    iout = jnp.zeros((T, K), jnp.int32)
    pout = jnp.zeros((T, K), jnp.float32)
    for kk in range(K):
        iout = jnp.where(kl == kk, idxs[kk], iout)
        pout = jnp.where(kl == kk, exps[kk] / denom, pout)
    idx_ref[...] = iout
    prob_ref[...] = pout


def _gating(x_flat, Wg, trust):
    T, D = x_flat.shape
    E = Wg.shape[0]
    return pl.pallas_call(
        _gating_body,
        out_shape=(jax.ShapeDtypeStruct((T, K), jnp.int32),
                   jax.ShapeDtypeStruct((T, K), jnp.float32)),
        in_specs=[
            pl.BlockSpec((T, D), lambda: (0, 0)),
            pl.BlockSpec((E, D), lambda: (0, 0)),
            pl.BlockSpec((1, E), lambda: (0, 0)),
        ],
        out_specs=(pl.BlockSpec((T, K), lambda: (0, 0)),
                   pl.BlockSpec((T, K), lambda: (0, 0))),
    )(x_flat, Wg, trust.reshape(1, E))


def _dispatch(topk_idx, topk_probs, E, nblk):
    T = topk_idx.shape[0]
    TK = T * K
    flat_e = topk_idx.reshape(-1)
    order = jnp.argsort(flat_e, stable=True).astype(jnp.int32)
    sorted_tok = (order // K).astype(jnp.int32)
    sorted_prob = topk_probs.reshape(-1)[order]
    counts = jnp.bincount(flat_e, length=E).astype(jnp.int32)
    offs = jnp.cumsum(counts) - counts
    nb = (counts + BT - 1) // BT
    cnb = jnp.cumsum(nb)
    bloff = cnb - nb
    bi = jnp.arange(nblk, dtype=jnp.int32)
    eid = jnp.searchsorted(cnb, bi, side='right').astype(jnp.int32)
    eid = jnp.minimum(eid, E - 1)
    j = bi - bloff[eid]
    start = offs[eid] + j * BT
    blen = jnp.clip(counts[eid] - j * BT, 0, BT)
    s = jnp.arange(nblk * BT, dtype=jnp.int32)
    sb, r = s // BT, s % BT
    pos = jnp.minimum(start[sb] + r, TK - 1)
    valid = r < blen[sb]
    slot_tok = jnp.where(valid, sorted_tok[pos], 0).astype(jnp.int32)
    slot_prob = jnp.where(valid, sorted_prob[pos], 0.0)
    return eid, blen.astype(jnp.int32), slot_tok, slot_prob


def _gelu(h):
    return 0.5 * h * (1.0 + lax.erf(h * (1.0 / math.sqrt(2.0))))


def _first_of_expert(b, er):
    return jnp.logical_or(b == 0, er[b] != er[jnp.maximum(b - 1, 0)])


def _ffn_body(eid_ref, blen_ref, xbf_ref, w1_ref, b1_ref, w2_ref, b2_ref,
              tokr_ref, tokc_ref, prob_ref, out_ref,
              w1b_ref, w2b_ref, xb_ref, eo_ref, acc_ref):
    b = pl.program_id(0)
    hcp = pl.program_id(1)
    nblk = pl.num_programs(0) - EPI
    T = acc_ref.shape[0]

    @pl.when(jnp.logical_and(b == 0, hcp == 0))
    def _():
        acc_ref[...] = jnp.zeros_like(acc_ref)

    active = jnp.logical_and(b < nblk, blen_ref[jnp.minimum(b, nblk - 1)] > 0)
    new_e = _first_of_expert(b, eid_ref)

    @pl.when(jnp.logical_and(active, new_e))
    def _():
        w1b_ref[hcp] = w1_ref[0].astype(jnp.bfloat16)
        w2b_ref[hcp] = w2_ref[0].astype(jnp.bfloat16)

    @pl.when(jnp.logical_and(active, hcp == 0))
    def _():
        tok_col = tokc_ref[0]
        diff_g = (lax.broadcasted_iota(jnp.int32, (BT, T), 1) - tok_col)
        onehot_g = (1.0 - jnp.minimum(
            jnp.abs(diff_g).astype(jnp.float32), 1.0)).astype(jnp.bfloat16)
        xb_ref[...] = lax.dot_general(
            onehot_g, xbf_ref[...], (((1,), (0,)), ((), ())),
            preferred_element_type=jnp.float32).astype(jnp.bfloat16)

    @pl.when(active)
    def _():
        xb = xb_ref[...]
        w1c = w1b_ref[hcp]
        h = lax.dot_general(xb, w1c, (((1,), (1,)), ((), ())),
                            preferred_element_type=jnp.float32)
        h = h + b1_ref[0, 0, 0]
        h = _gelu(h).astype(jnp.bfloat16)
        w2c = w2b_ref[hcp]
        p = lax.dot_general(h, w2c, (((1,), (1,)), ((), ())),
                            preferred_element_type=jnp.float32)

        @pl.when(hcp == 0)
        def _():
            eo_ref[...] = p

        @pl.when(hcp != 0)
        def _():
            eo_ref[...] += p

    @pl.when(jnp.logical_and(active, hcp == NHC - 1))
    def _():
        eo = eo_ref[...] + b2_ref[0, 0]
        prob_col = prob_ref[0]
        contrib = (prob_col * eo).astype(jnp.bfloat16)
        tok_row = tokr_ref[0]
        tok_b = tok_row + jnp.zeros((T, 1), jnp.int32)
        diff_s = lax.broadcasted_iota(jnp.int32, tok_b.shape, 0) - tok_b
        onehot_t = (1.0 - jnp.minimum(
            jnp.abs(diff_s).astype(jnp.float32), 1.0)).astype(jnp.bfloat16)
        acc_ref[...] += lax.dot_general(
            onehot_t, contrib, (((1,), (0,)), ((), ())),
            preferred_element_type=jnp.float32)

    @pl.when(jnp.logical_and(b >= nblk, hcp == NHC - 1))
    def _():
        out_ref[...] = acc_ref[pl.ds((b - nblk) * (T // EPI), T // EPI), :]


def _ffn(xbf, W1, b1, W2, b2, eid, blen, slot_tok, slot_prob, T, nblk):
    E, H, D = W1.shape
    nsteps = nblk + EPI

    def w1_map(b, hcp, er, lr):
        return (er[b], jnp.where(_first_of_expert(b, er), hcp, NHC - 1), 0)

    def w2_map(b, hcp, er, lr):
        return (er[b], 0, jnp.where(_first_of_expert(b, er), hcp, NHC - 1))

    def blk_map(b, hcp, er, lr):
        return (jnp.minimum(b, er.shape[0] - EPI - 1), 0, 0)

    grid_spec = pltpu.PrefetchScalarGridSpec(
        num_scalar_prefetch=2,
        grid=(nsteps, NHC),
        in_specs=[
            pl.BlockSpec((T, D), lambda b, hcp, er, lr: (0, 0)),
            pl.BlockSpec((1, HC, D), w1_map),
            pl.BlockSpec((1, 1, 1, HC), lambda b, hcp, er, lr: (er[b], hcp, 0, 0)),
            pl.BlockSpec((1, D, HC), w2_map),
            pl.BlockSpec((1, 1, D), lambda b, hcp, er, lr: (er[b], 0, 0)),
            pl.BlockSpec((1, 1, BT), blk_map),
            pl.BlockSpec((1, BT, 1), blk_map),
            pl.BlockSpec((1, BT, 1), blk_map),
        ],
        out_specs=pl.BlockSpec(
            (T // EPI, D),
            lambda b, hcp, er, lr: (jnp.clip(b - (er.shape[0] - EPI), 0, EPI - 1), 0)),
        scratch_shapes=[pltpu.VMEM((NHC, HC, D), jnp.bfloat16),
                        pltpu.VMEM((NHC, D, HC), jnp.bfloat16),
                        pltpu.VMEM((BT, D), jnp.bfloat16),
                        pltpu.VMEM((BT, D), jnp.float32),
                        pltpu.VMEM((T, D), jnp.float32)],
    )
    eid_pad = jnp.concatenate(
        [eid, jnp.full((EPI,), E - 1, jnp.int32)])
    return pl.pallas_call(
        _ffn_body,
        grid_spec=grid_spec,
        out_shape=jax.ShapeDtypeStruct((T, D), jnp.float32),
        compiler_params=pltpu.CompilerParams(
            dimension_semantics=("arbitrary", "arbitrary"),
        ),
    )(eid_pad, blen, xbf, W1, b1.reshape(E, NHC, 1, HC), W2,
      b2.reshape(E, 1, D), slot_tok.reshape(nblk, 1, BT),
      slot_tok.reshape(nblk, BT, 1), slot_prob.reshape(nblk, BT, 1))


def kernel(x, Wg, W1, b1, W2, b2, trust_scores):
    Bq, Sq, Dq = x.shape
    x_flat = x.reshape(-1, Dq)
    T = x_flat.shape[0]
    E, H, D = W1.shape
    nblk = (T * K) // BT + E

    topk_idx, topk_probs = _gating(x_flat, Wg, trust_scores)
    eid, blen, slot_tok, slot_prob = _dispatch(topk_idx, topk_probs, E, nblk)
    out = _ffn(x_flat.astype(jnp.bfloat16), W1, b1, W2, b2,
               eid, blen, slot_tok, slot_prob, T, nblk)
    return out.reshape(Bq, Sq, Dq)


# speedup vs baseline: 3.2009x; 1.0096x over previous
import math

import jax
import jax.numpy as jnp
from jax import lax
from jax.experimental import pallas as pl
from jax.experimental.pallas import tpu as pltpu

K = 4
BT = 256
HC = 1024
NHC = 4
EPI = 4


def _gating_body(x_ref, wg_ref, trust_ref, idx_ref, prob_ref):
    T = x_ref.shape[0]
    E = wg_ref.shape[0]
    gates = lax.dot_general(
        x_ref[...], wg_ref[...], (((1,), (1,)), ((), ())),
        preferred_element_type=jnp.float32)
    trust = trust_ref[0, :]
    sig = 1.0 / (1.0 + jnp.exp(-trust))
    cur = gates * sig
    lane = lax.broadcasted_iota(jnp.int32, (T, E), 1)
    vals, idxs = [], []
    for _ in range(K):
        m = jnp.max(cur, axis=1, keepdims=True)
        idx = jnp.min(jnp.where(cur == m, lane, E), axis=1, keepdims=True)
        vals.append(m)
        idxs.append(idx)
        cur = jnp.where(lane == idx, -1e30, cur)
    v0 = vals[0]
    exps = [jnp.exp(v - v0) for v in vals]
    denom = exps[0] + exps[1] + exps[2] + exps[3]
    kl = lax.broadcasted_iota(jnp.int32, (T, K), 1)
    iout = jnp.zeros((T, K), jnp.int32)
    pout = jnp.zeros((T, K), jnp.float32)
    for kk in range(K):
        iout = jnp.where(kl == kk, idxs[kk], iout)
        pout = jnp.where(kl == kk, exps[kk] / denom, pout)
    idx_ref[...] = iout
    prob_ref[...] = pout


def _gating(x_flat, Wg, trust):
    T, D = x_flat.shape
    E = Wg.shape[0]
    return pl.pallas_call(
        _gating_body,
        out_shape=(jax.ShapeDtypeStruct((T, K), jnp.int32),
                   jax.ShapeDtypeStruct((T, K), jnp.float32)),
        in_specs=[
            pl.BlockSpec((T, D), lambda: (0, 0)),
            pl.BlockSpec((E, D), lambda: (0, 0)),
            pl.BlockSpec((1, E), lambda: (0, 0)),
        ],
        out_specs=(pl.BlockSpec((T, K), lambda: (0, 0)),
                   pl.BlockSpec((T, K), lambda: (0, 0))),
    )(x_flat, Wg, trust.reshape(1, E))


def _dispatch(topk_idx, topk_probs, E, nblk):
    T = topk_idx.shape[0]
    TK = T * K
    flat_e = topk_idx.reshape(-1)
    keys = flat_e * TK + jnp.arange(TK, dtype=jnp.int32)
    order_sorted = jnp.sort(keys) % TK
    counts = jnp.bincount(flat_e, length=E).astype(jnp.int32)
    offs = jnp.cumsum(counts) - counts
    nb = (counts + BT - 1) // BT
    cnb = jnp.cumsum(nb)
    bloff = cnb - nb
    bi = jnp.arange(nblk, dtype=jnp.int32)
    eid = jnp.searchsorted(cnb, bi, side='right').astype(jnp.int32)
    eid = jnp.minimum(eid, E - 1)
    j = bi - bloff[eid]
    start = offs[eid] + j * BT
    blen = jnp.clip(counts[eid] - j * BT, 0, BT)
    s = jnp.arange(nblk * BT, dtype=jnp.int32)
    sb, r = s // BT, s % BT
    pos = jnp.minimum(start[sb] + r, TK - 1)
    valid = r < blen[sb]
    g = order_sorted[pos]
    slot_tok = jnp.where(valid, g // K, 0).astype(jnp.int32)
    slot_prob = jnp.where(valid, topk_probs.reshape(-1)[g], 0.0)
    return eid, blen.astype(jnp.int32), slot_tok, slot_prob


def _gelu(h):
    return 0.5 * h * (1.0 + lax.erf(h * (1.0 / math.sqrt(2.0))))


def _first_of_expert(b, er):
    return jnp.logical_or(b == 0, er[b] != er[jnp.maximum(b - 1, 0)])


def _ffn_body(eid_ref, blen_ref, xbf_ref, w1_ref, b1_ref, w2_ref, b2_ref,
              tokr_ref, tokc_ref, prob_ref, out_ref,
              w1b_ref, w2b_ref, xb_ref, eo_ref, acc_ref):
    b = pl.program_id(0)
    hcp = pl.program_id(1)
    nblk = pl.num_programs(0) - EPI
    T = acc_ref.shape[0]

    @pl.when(jnp.logical_and(b == 0, hcp == 0))
    def _():
        acc_ref[...] = jnp.zeros_like(acc_ref)

    active = jnp.logical_and(b < nblk, blen_ref[jnp.minimum(b, nblk - 1)] > 0)
    new_e = _first_of_expert(b, eid_ref)

    @pl.when(jnp.logical_and(active, new_e))
    def _():
        w1b_ref[hcp] = w1_ref[0].astype(jnp.bfloat16)
        w2b_ref[hcp] = w2_ref[0].astype(jnp.bfloat16)

    @pl.when(jnp.logical_and(active, hcp == 0))
    def _():
        tok_col = tokc_ref[0]
        diff_g = (lax.broadcasted_iota(jnp.int32, (BT, T), 1) - tok_col)
        onehot_g = (1.0 - jnp.minimum(
            jnp.abs(diff_g).astype(jnp.float32), 1.0)).astype(jnp.bfloat16)
        xb_ref[...] = lax.dot_general(
            onehot_g, xbf_ref[...], (((1,), (0,)), ((), ())),
            preferred_element_type=jnp.float32).astype(jnp.bfloat16)

    @pl.when(active)
    def _():
        xb = xb_ref[...]
        w1c = w1b_ref[hcp]
        h = lax.dot_general(xb, w1c, (((1,), (1,)), ((), ())),
                            preferred_element_type=jnp.float32)
        h = h + b1_ref[0, 0, 0]
        h = _gelu(h).astype(jnp.bfloat16)
        w2c = w2b_ref[hcp]
        p = lax.dot_general(h, w2c, (((1,), (1,)), ((), ())),
                            preferred_element_type=jnp.float32)

        @pl.when(hcp == 0)
        def _():
            eo_ref[...] = p

        @pl.when(hcp != 0)
        def _():
            eo_ref[...] += p

    @pl.when(jnp.logical_and(active, hcp == NHC - 1))
    def _():
        eo = eo_ref[...] + b2_ref[0, 0]
        prob_col = prob_ref[0]
        contrib = (prob_col * eo).astype(jnp.bfloat16)
        tok_row = tokr_ref[0]
        tok_b = tok_row + jnp.zeros((T, 1), jnp.int32)
        diff_s = lax.broadcasted_iota(jnp.int32, tok_b.shape, 0) - tok_b
        onehot_t = (1.0 - jnp.minimum(
            jnp.abs(diff_s).astype(jnp.float32), 1.0)).astype(jnp.bfloat16)
        acc_ref[...] += lax.dot_general(
            onehot_t, contrib, (((1,), (0,)), ((), ())),
            preferred_element_type=jnp.float32)

    @pl.when(jnp.logical_and(b >= nblk, hcp == NHC - 1))
    def _():
        out_ref[...] = acc_ref[pl.ds((b - nblk) * (T // EPI), T // EPI), :]


def _ffn(xbf, W1, b1, W2, b2, eid, blen, slot_tok, slot_prob, T, nblk):
    E, H, D = W1.shape
    nsteps = nblk + EPI

    def w1_map(b, hcp, er, lr):
        return (er[b], jnp.where(_first_of_expert(b, er), hcp, NHC - 1), 0)

    def w2_map(b, hcp, er, lr):
        return (er[b], 0, jnp.where(_first_of_expert(b, er), hcp, NHC - 1))

    def blk_map(b, hcp, er, lr):
        return (jnp.minimum(b, er.shape[0] - EPI - 1), 0, 0)

    grid_spec = pltpu.PrefetchScalarGridSpec(
        num_scalar_prefetch=2,
        grid=(nsteps, NHC),
        in_specs=[
            pl.BlockSpec((T, D), lambda b, hcp, er, lr: (0, 0)),
            pl.BlockSpec((1, HC, D), w1_map),
            pl.BlockSpec((1, 1, 1, HC), lambda b, hcp, er, lr: (er[b], hcp, 0, 0)),
            pl.BlockSpec((1, D, HC), w2_map),
            pl.BlockSpec((1, 1, D), lambda b, hcp, er, lr: (er[b], 0, 0)),
            pl.BlockSpec((1, 1, BT), blk_map),
            pl.BlockSpec((1, BT, 1), blk_map),
            pl.BlockSpec((1, BT, 1), blk_map),
        ],
        out_specs=pl.BlockSpec(
            (T // EPI, D),
            lambda b, hcp, er, lr: (jnp.clip(b - (er.shape[0] - EPI), 0, EPI - 1), 0)),
        scratch_shapes=[pltpu.VMEM((NHC, HC, D), jnp.bfloat16),
                        pltpu.VMEM((NHC, D, HC), jnp.bfloat16),
                        pltpu.VMEM((BT, D), jnp.bfloat16),
                        pltpu.VMEM((BT, D), jnp.float32),
                        pltpu.VMEM((T, D), jnp.float32)],
    )
    eid_pad = jnp.concatenate(
        [eid, jnp.full((EPI,), E - 1, jnp.int32)])
    return pl.pallas_call(
        _ffn_body,
        grid_spec=grid_spec,
        out_shape=jax.ShapeDtypeStruct((T, D), jnp.float32),
        compiler_params=pltpu.CompilerParams(
            dimension_semantics=("arbitrary", "arbitrary"),
        ),
    )(eid_pad, blen, xbf, W1, b1.reshape(E, NHC, 1, HC), W2,
      b2.reshape(E, 1, D), slot_tok.reshape(nblk, 1, BT),
      slot_tok.reshape(nblk, BT, 1), slot_prob.reshape(nblk, BT, 1))


def kernel(x, Wg, W1, b1, W2, b2, trust_scores):
    Bq, Sq, Dq = x.shape
    x_flat = x.reshape(-1, Dq)
    T = x_flat.shape[0]
    E, H, D = W1.shape
    nblk = (T * K) // BT + E

    topk_idx, topk_probs = _gating(x_flat, Wg, trust_scores)
    eid, blen, slot_tok, slot_prob = _dispatch(topk_idx, topk_probs, E, nblk)
    out = _ffn(x_flat.astype(jnp.bfloat16), W1, b1, W2, b2,
               eid, blen, slot_tok, slot_prob, T, nblk)
    return out.reshape(Bq, Sq, Dq)
